# baseline (speedup 1.0000x reference)
import functools

import jax
import jax.numpy as jnp
from jax import lax
from jax.experimental import pallas as pl
from jax.experimental.pallas import tpu as pltpu
from jax.experimental.pallas import tpu_sc as plsc

NC = 2
NS = 16
NW = NC * NS
LANES = 16
CHUNK = 400
NBUF = 4
SC_B = 1024
TC_BB = 8


def _sc_decode_call(xf, alphabet_codes, N, V, row0, rows_sc):
    rows_per_w = rows_sc // NW
    chunks = rows_per_w // CHUNK

    mesh = plsc.VectorSubcoreMesh(
        core_axis_name="c", subcore_axis_name="s",
        num_cores=NC, num_subcores=NS)

    @functools.partial(
        pl.kernel,
        out_type=jax.ShapeDtypeStruct((rows_sc,), jnp.int32),
        mesh=mesh,
        scratch_types=(
            [pltpu.VMEM((CHUNK * V,), jnp.float32) for _ in range(NBUF)]
            + [pltpu.VMEM((CHUNK,), jnp.int32),
               pltpu.VMEM((V,), jnp.int32)]
            + [pltpu.SemaphoreType.DMA for _ in range(NBUF)]
        ),
        compiler_params=pltpu.CompilerParams(needs_layout_passes=False),
    )
    def sc_decode(x_hbm, alpha_hbm, out_hbm, *refs):
        bufs = refs[:NBUF]
        obuf, alpha_v = refs[NBUF], refs[NBUF + 1]
        sems = refs[NBUF + 2:]

        wid = lax.axis_index("s") * NC + lax.axis_index("c")
        base = wid * rows_per_w

        def in_slice(g):
            return x_hbm.at[pl.ds((row0 + base + g * CHUNK) * V, CHUNK * V)]

        pltpu.sync_copy(alpha_hbm, alpha_v)
        for b in range(NBUF):
            pltpu.async_copy(in_slice(b), bufs[b], sems[b])

        lane = lax.iota(jnp.int32, LANES)
        ib = [lane + 16 * q for q in range(4)]
        lane_is = [lane == j for j in range(LANES)]

        def chunk_body(g, b):
            pltpu.make_async_copy(in_slice(g), bufs[b], sems[b]).wait()
            bb = bufs[b]

            def row(r):
                w = r * V
                v0 = bb[pl.ds(w, LANES)]
                v1 = bb[pl.ds(w + 16, LANES)]
                v2 = bb[pl.ds(w + 32, LANES)]
                v3 = bb[pl.ds(w + 48, LANES)]
                u = v1 > v0
                m01 = jnp.where(u, v1, v0)
                i01 = jnp.where(u, ib[1], ib[0])
                u = v3 > v2
                m23 = jnp.where(u, v3, v2)
                i23 = jnp.where(u, ib[3], ib[2])
                u = m23 > m01
                m = jnp.where(u, m23, m01)
                i = jnp.where(u, i23, i01)
                cand = jnp.where(m == jnp.max(m), i, V)
                return jnp.min(cand)

            def group(gr, carry):
                r0 = gr * LANES
                acc = jnp.zeros((LANES,), jnp.int32)
                for j in range(LANES):
                    acc = jnp.where(lane_is[j], row(r0 + j), acc)
                obuf[pl.ds(r0, LANES)] = plsc.load_gather(alpha_v, [acc])
                return carry

            lax.fori_loop(0, CHUNK // LANES, group, 0)
            pltpu.sync_copy(obuf, out_hbm.at[pl.ds(base + g * CHUNK, CHUNK)])

            nxt = g + NBUF

            @pl.when(nxt < chunks)
            def _():
                pltpu.async_copy(in_slice(nxt), bufs[b], sems[b])

        def ring_body(i, carry):
            for b in range(NBUF):
                chunk_body(i * NBUF + b, b)
            return carry

        lax.fori_loop(0, chunks // NBUF, ring_body, 0)

    return sc_decode(xf, alphabet_codes)


def _tc_decode_call(x, comb, tc_b, T, V):
    def tc_kernel(comb_ref, x_ref, o_ref):
        xb = x_ref[...]
        m = jnp.max(xb, axis=-1, keepdims=True)
        key = jnp.where(xb == m, comb_ref[...][None, None, :],
                        jnp.int32(1 << 30))
        o_ref[...] = jnp.min(key, axis=-1) & 255

    return pl.pallas_call(
        tc_kernel,
        grid=(tc_b // TC_BB,),
        in_specs=[
            pl.BlockSpec((V,), lambda i: (0,)),
            pl.BlockSpec((TC_BB, T, V), lambda i: (i, 0, 0)),
        ],
        out_specs=pl.BlockSpec((TC_BB, T), lambda i: (i, 0)),
        out_shape=jax.ShapeDtypeStruct((tc_b, T), jnp.int32),
    )(comb, x)


def kernel(x, alphabet_codes):
    B, T, V = x.shape
    N = B * T
    tc_b = B - SC_B
    xf = x.reshape(N * V)
    comb = (jnp.arange(V, dtype=jnp.int32) << 8) | alphabet_codes

    sc_out = _sc_decode_call(xf, alphabet_codes, N, V, tc_b * T, SC_B * T)
    tc_out = _tc_decode_call(x, comb, tc_b, T, V)
    return jnp.concatenate([tc_out, sc_out.reshape(SC_B, T)], axis=0)


# speedup vs baseline: 1.0016x; 1.0016x over previous
import functools

import jax
import jax.numpy as jnp
from jax import lax
from jax.experimental import pallas as pl
from jax.experimental.pallas import tpu as pltpu
from jax.experimental.pallas import tpu_sc as plsc

NC = 2
NS = 16
NW = NC * NS
LANES = 16
CHUNK = 400
NBUF = 4
SC_B = 1024
TC_BB = 8


def _sc_decode_call(xf, alphabet_codes, N, V, row0, rows_sc):
    rows_per_w = rows_sc // NW
    chunks = rows_per_w // CHUNK

    mesh = plsc.VectorSubcoreMesh(
        core_axis_name="c", subcore_axis_name="s",
        num_cores=NC, num_subcores=NS)

    @functools.partial(
        pl.kernel,
        out_type=jax.ShapeDtypeStruct((rows_sc,), jnp.int32),
        mesh=mesh,
        scratch_types=(
            [pltpu.VMEM((CHUNK * V,), jnp.float32) for _ in range(NBUF)]
            + [pltpu.VMEM((CHUNK,), jnp.int32),
               pltpu.VMEM((V,), jnp.int32)]
            + [pltpu.SemaphoreType.DMA for _ in range(NBUF)]
        ),
        compiler_params=pltpu.CompilerParams(
            needs_layout_passes=False, use_tc_tiling_on_sc=True),
    )
    def sc_decode(x_hbm, alpha_hbm, out_hbm, *refs):
        bufs = refs[:NBUF]
        obuf, alpha_v = refs[NBUF], refs[NBUF + 1]
        sems = refs[NBUF + 2:]

        wid = lax.axis_index("s") * NC + lax.axis_index("c")
        base = wid * rows_per_w

        def in_slice(g):
            return x_hbm.at[pl.ds((row0 + base + g * CHUNK) * V, CHUNK * V)]

        pltpu.sync_copy(alpha_hbm, alpha_v)
        for b in range(NBUF):
            pltpu.async_copy(in_slice(b), bufs[b], sems[b])

        lane = lax.iota(jnp.int32, LANES)
        ib = [lane + 16 * q for q in range(4)]
        lane_is = [lane == j for j in range(LANES)]

        def chunk_body(g, b):
            pltpu.make_async_copy(in_slice(g), bufs[b], sems[b]).wait()
            bb = bufs[b]

            def row(r):
                w = r * V
                v0 = bb[pl.ds(w, LANES)]
                v1 = bb[pl.ds(w + 16, LANES)]
                v2 = bb[pl.ds(w + 32, LANES)]
                v3 = bb[pl.ds(w + 48, LANES)]
                u = v1 > v0
                m01 = jnp.where(u, v1, v0)
                i01 = jnp.where(u, ib[1], ib[0])
                u = v3 > v2
                m23 = jnp.where(u, v3, v2)
                i23 = jnp.where(u, ib[3], ib[2])
                u = m23 > m01
                m = jnp.where(u, m23, m01)
                i = jnp.where(u, i23, i01)
                cand = jnp.where(m == jnp.max(m), i, V)
                return jnp.min(cand)

            def group(gr, carry):
                r0 = gr * LANES
                acc = jnp.zeros((LANES,), jnp.int32)
                for j in range(LANES):
                    acc = jnp.where(lane_is[j], row(r0 + j), acc)
                obuf[pl.ds(r0, LANES)] = plsc.load_gather(alpha_v, [acc])
                return carry

            lax.fori_loop(0, CHUNK // LANES, group, 0)
            pltpu.sync_copy(obuf, out_hbm.at[pl.ds(base + g * CHUNK, CHUNK)])

            nxt = g + NBUF

            @pl.when(nxt < chunks)
            def _():
                pltpu.async_copy(in_slice(nxt), bufs[b], sems[b])

        def ring_body(i, carry):
            for b in range(NBUF):
                chunk_body(i * NBUF + b, b)
            return carry

        lax.fori_loop(0, chunks // NBUF, ring_body, 0)

    return sc_decode(xf, alphabet_codes)


def _tc_decode_call(x, comb, tc_b, T, V):
    def tc_kernel(comb_ref, x_ref, o_ref):
        xb = x_ref[...]
        m = jnp.max(xb, axis=-1, keepdims=True)
        key = jnp.where(xb == m, comb_ref[...][None, None, :],
                        jnp.int32(1 << 30))
        o_ref[...] = jnp.min(key, axis=-1) & 255

    return pl.pallas_call(
        tc_kernel,
        grid=(tc_b // TC_BB,),
        in_specs=[
            pl.BlockSpec((V,), lambda i: (0,)),
            pl.BlockSpec((TC_BB, T, V), lambda i: (i, 0, 0)),
        ],
        out_specs=pl.BlockSpec((TC_BB, T), lambda i: (i, 0)),
        out_shape=jax.ShapeDtypeStruct((tc_b, T), jnp.int32),
    )(comb, x)


def kernel(x, alphabet_codes):
    B, T, V = x.shape
    N = B * T
    tc_b = B - SC_B
    xf = x.reshape(N * V)
    comb = (jnp.arange(V, dtype=jnp.int32) << 8) | alphabet_codes

    sc_out = _sc_decode_call(xf, alphabet_codes, N, V, tc_b * T, SC_B * T)
    tc_out = _tc_decode_call(x, comb, tc_b, T, V)
    return jnp.concatenate([tc_out, sc_out.reshape(SC_B, T)], axis=0)


# speedup vs baseline: 1.4419x; 1.4395x over previous
import functools

import jax
import jax.numpy as jnp
from jax import lax
from jax.experimental import pallas as pl
from jax.experimental.pallas import tpu as pltpu
from jax.experimental.pallas import tpu_sc as plsc

NC = 2
NS = 16
NW = NC * NS
LANES = 16
CHUNK = 320
NBUF = 2
SC_B = 1024
TC_BB = 8


def _sc_decode_call(x2, alphabet_codes, V, row0, rows_sc):
    rows_per_w = rows_sc // NW
    chunks = rows_per_w // CHUNK

    mesh = plsc.VectorSubcoreMesh(
        core_axis_name="c", subcore_axis_name="s",
        num_cores=NC, num_subcores=NS)

    @functools.partial(
        pl.kernel,
        out_type=jax.ShapeDtypeStruct((rows_sc,), jnp.int32),
        mesh=mesh,
        scratch_types=(
            [pltpu.VMEM((CHUNK, V), jnp.float32) for _ in range(NBUF)]
            + [pltpu.VMEM((CHUNK,), jnp.int32),
               pltpu.VMEM((V,), jnp.int32)]
            + [pltpu.SemaphoreType.DMA for _ in range(NBUF)]
        ),
        compiler_params=pltpu.CompilerParams(
            needs_layout_passes=False, use_tc_tiling_on_sc=True),
    )
    def sc_decode(x_hbm, alpha_hbm, out_hbm, *refs):
        bufs = refs[:NBUF]
        obuf, alpha_v = refs[NBUF], refs[NBUF + 1]
        sems = refs[NBUF + 2:]

        wid = lax.axis_index("s") * NC + lax.axis_index("c")
        base = wid * rows_per_w

        def in_slice(g):
            return x_hbm.at[pl.ds(row0 + base + g * CHUNK, CHUNK)]

        pltpu.sync_copy(alpha_hbm, alpha_v)
        for b in range(NBUF):
            pltpu.async_copy(in_slice(b), bufs[b], sems[b])

        lane = lax.iota(jnp.int32, LANES)
        ib = [lane + 16 * q for q in range(4)]
        lane_is = [lane == j for j in range(LANES)]

        def chunk_body(g, b):
            pltpu.make_async_copy(in_slice(g), bufs[b], sems[b]).wait()
            bb = bufs[b]

            def row(r):
                v0 = bb[r, pl.ds(0, LANES)]
                v1 = bb[r, pl.ds(16, LANES)]
                v2 = bb[r, pl.ds(32, LANES)]
                v3 = bb[r, pl.ds(48, LANES)]
                u = v1 > v0
                m01 = jnp.where(u, v1, v0)
                i01 = jnp.where(u, ib[1], ib[0])
                u = v3 > v2
                m23 = jnp.where(u, v3, v2)
                i23 = jnp.where(u, ib[3], ib[2])
                u = m23 > m01
                m = jnp.where(u, m23, m01)
                i = jnp.where(u, i23, i01)
                cand = jnp.where(m == jnp.max(m), i, V)
                return jnp.min(cand)

            def group(gr, carry):
                r0 = gr * LANES
                acc = jnp.zeros((LANES,), jnp.int32)
                for j in range(LANES):
                    acc = jnp.where(lane_is[j], row(r0 + j), acc)
                obuf[pl.ds(r0, LANES)] = plsc.load_gather(alpha_v, [acc])
                return carry

            lax.fori_loop(0, CHUNK // LANES, group, 0)
            pltpu.sync_copy(obuf, out_hbm.at[pl.ds(base + g * CHUNK, CHUNK)])

            nxt = g + NBUF

            @pl.when(nxt < chunks)
            def _():
                pltpu.async_copy(in_slice(nxt), bufs[b], sems[b])

        def ring_body(i, carry):
            for b in range(NBUF):
                chunk_body(i * NBUF + b, b)
            return carry

        lax.fori_loop(0, chunks // NBUF, ring_body, 0)

    return sc_decode(x2, alphabet_codes)


def _tc_decode_call(x, comb, tc_b, T, V):
    def tc_kernel(comb_ref, x_ref, o_ref):
        xb = x_ref[...]
        m = jnp.max(xb, axis=-1, keepdims=True)
        key = jnp.where(xb == m, comb_ref[...][None, None, :],
                        jnp.int32(1 << 30))
        o_ref[...] = jnp.min(key, axis=-1) & 255

    return pl.pallas_call(
        tc_kernel,
        grid=(tc_b // TC_BB,),
        in_specs=[
            pl.BlockSpec((V,), lambda i: (0,)),
            pl.BlockSpec((TC_BB, T, V), lambda i: (i, 0, 0)),
        ],
        out_specs=pl.BlockSpec((TC_BB, T), lambda i: (i, 0)),
        out_shape=jax.ShapeDtypeStruct((tc_b, T), jnp.int32),
    )(comb, x)


def kernel(x, alphabet_codes):
    B, T, V = x.shape
    N = B * T
    tc_b = B - SC_B
    x2 = x.reshape(N, V)
    comb = (jnp.arange(V, dtype=jnp.int32) << 8) | alphabet_codes

    sc_out = _sc_decode_call(x2, alphabet_codes, V, tc_b * T, SC_B * T)
    tc_out = _tc_decode_call(x, comb, tc_b, T, V)
    return jnp.concatenate([tc_out, sc_out.reshape(SC_B, T)], axis=0)


# speedup vs baseline: 1.5747x; 1.0921x over previous
import functools

import jax
import jax.numpy as jnp
from jax import lax
from jax.experimental import pallas as pl
from jax.experimental.pallas import tpu as pltpu
from jax.experimental.pallas import tpu_sc as plsc

NC = 2
NS = 16
NW = NC * NS
LANES = 16
CHUNK = 320
NBUF = 2
SC_B = 1024
TC_BB = 8


def _sc_decode_call(x2, alphabet_codes, V, row0, rows_sc):
    rows_per_w = rows_sc // NW
    chunks = rows_per_w // CHUNK

    mesh = plsc.VectorSubcoreMesh(
        core_axis_name="c", subcore_axis_name="s",
        num_cores=NC, num_subcores=NS)

    @functools.partial(
        pl.kernel,
        out_type=jax.ShapeDtypeStruct((rows_sc,), jnp.int32),
        mesh=mesh,
        scratch_types=(
            [pltpu.VMEM((CHUNK, V), jnp.float32) for _ in range(NBUF)]
            + [pltpu.VMEM((CHUNK,), jnp.int32),
               pltpu.VMEM((V,), jnp.int32)]
            + [pltpu.SemaphoreType.DMA for _ in range(NBUF)]
        ),
        compiler_params=pltpu.CompilerParams(
            needs_layout_passes=False, use_tc_tiling_on_sc=True),
    )
    def sc_decode(x_hbm, alpha_hbm, out_hbm, *refs):
        bufs = refs[:NBUF]
        obuf, alpha_v = refs[NBUF], refs[NBUF + 1]
        sems = refs[NBUF + 2:]

        wid = lax.axis_index("s") * NC + lax.axis_index("c")
        base = wid * rows_per_w

        def in_slice(g):
            return x_hbm.at[pl.ds(row0 + base + g * CHUNK, CHUNK)]

        pltpu.sync_copy(alpha_hbm, alpha_v)
        for b in range(NBUF):
            pltpu.async_copy(in_slice(b), bufs[b], sems[b])

        lane = lax.iota(jnp.int32, LANES)
        ib = [lane + 16 * q for q in range(4)]
        lane_is = [lane == j for j in range(LANES)]

        def chunk_body(g, b):
            pltpu.make_async_copy(in_slice(g), bufs[b], sems[b]).wait()
            bb = bufs[b]

            def row(r):
                v0 = bb[r, pl.ds(0, LANES)]
                v1 = bb[r, pl.ds(16, LANES)]
                v2 = bb[r, pl.ds(32, LANES)]
                v3 = bb[r, pl.ds(48, LANES)]
                u = v1 > v0
                m01 = jnp.where(u, v1, v0)
                i01 = jnp.where(u, ib[1], ib[0])
                u = v3 > v2
                m23 = jnp.where(u, v3, v2)
                i23 = jnp.where(u, ib[3], ib[2])
                u = m23 > m01
                m = jnp.where(u, m23, m01)
                i = jnp.where(u, i23, i01)
                cand = jnp.where(m == jnp.max(m), i, V)
                return jnp.min(cand)

            def group(gr, carry):
                r0 = gr * LANES
                acc = jnp.zeros((LANES,), jnp.int32)
                for j in range(LANES):
                    acc = jnp.where(lane_is[j], row(r0 + j), acc)
                obuf[pl.ds(r0, LANES)] = plsc.load_gather(alpha_v, [acc])
                return carry

            lax.fori_loop(0, CHUNK // LANES, group, 0)
            pltpu.sync_copy(obuf, out_hbm.at[pl.ds(base + g * CHUNK, CHUNK)])

            nxt = g + NBUF

            @pl.when(nxt < chunks)
            def _():
                pltpu.async_copy(in_slice(nxt), bufs[b], sems[b])

        def ring_body(i, carry):
            for b in range(NBUF):
                chunk_body(i * NBUF + b, b)
            return carry

        lax.fori_loop(0, chunks // NBUF, ring_body, 0)

    return sc_decode(x2, alphabet_codes)


def _tc_decode_call(x, comb, tc_b, T, V):
    def tc_kernel(comb_ref, x_ref, o_ref):
        xb = x_ref[...]
        m = jnp.max(xb, axis=-1, keepdims=True)
        keyf = jnp.where(xb == m, comb_ref[...][None, None, :],
                         -jnp.inf)
        best = jnp.max(keyf, axis=-1)
        o_ref[...] = (-best).astype(jnp.int32) & 255

    return pl.pallas_call(
        tc_kernel,
        grid=(tc_b // TC_BB,),
        in_specs=[
            pl.BlockSpec((V,), lambda i: (0,)),
            pl.BlockSpec((TC_BB, T, V), lambda i: (i, 0, 0)),
        ],
        compiler_params=pltpu.CompilerParams(
            dimension_semantics=("parallel",)),
        out_specs=pl.BlockSpec((TC_BB, T), lambda i: (i, 0)),
        out_shape=jax.ShapeDtypeStruct((tc_b, T), jnp.int32),
    )(comb, x)


def kernel(x, alphabet_codes):
    B, T, V = x.shape
    N = B * T
    tc_b = B - SC_B
    x2 = x.reshape(N, V)
    comb = -((jnp.arange(V, dtype=jnp.int32) << 8) | alphabet_codes
             ).astype(jnp.float32)

    sc_out = _sc_decode_call(x2, alphabet_codes, V, tc_b * T, SC_B * T)
    tc_out = _tc_decode_call(x, comb, tc_b, T, V)
    return jnp.concatenate([tc_out, sc_out.reshape(SC_B, T)], axis=0)


# speedup vs baseline: 6.8782x; 4.3679x over previous
import functools

import jax
import jax.numpy as jnp
from jax import lax
from jax.experimental import pallas as pl
from jax.experimental.pallas import tpu as pltpu
from jax.experimental.pallas import tpu_sc as plsc

NC = 2
NS = 16
NW = NC * NS
LANES = 16
TC_T = 152
TC_TB = 8
TC_BB = 512
SC_BB = 512
NBUF = 2


def _sc_decode_call(xT2, alphabet_codes, V, B, t0, sc_t):
    nchunks = sc_t * (B // SC_BB)
    per_w = nchunks // NW

    mesh = plsc.VectorSubcoreMesh(
        core_axis_name="c", subcore_axis_name="s",
        num_cores=NC, num_subcores=NS)

    @functools.partial(
        pl.kernel,
        out_type=jax.ShapeDtypeStruct((sc_t * B,), jnp.int32),
        mesh=mesh,
        scratch_types=(
            [pltpu.VMEM((V, SC_BB), jnp.float32) for _ in range(NBUF)]
            + [pltpu.VMEM((SC_BB,), jnp.int32),
               pltpu.VMEM((V,), jnp.int32)]
            + [pltpu.SemaphoreType.DMA for _ in range(NBUF)]
        ),
        compiler_params=pltpu.CompilerParams(
            needs_layout_passes=False, use_tc_tiling_on_sc=True),
    )
    def sc_decode(x_hbm, alpha_hbm, out_hbm, *refs):
        bufs = refs[:NBUF]
        obuf, alpha_v = refs[NBUF], refs[NBUF + 1]
        sems = refs[NBUF + 2:]

        wid = lax.axis_index("s") * NC + lax.axis_index("c")
        k0 = wid * per_w
        ncol = B // SC_BB

        def in_slice(k):
            t = t0 + k // ncol
            b0 = (k % ncol) * SC_BB
            return x_hbm.at[pl.ds(t * V, V), pl.ds(b0, SC_BB)]

        pltpu.sync_copy(alpha_hbm, alpha_v)
        for b in range(NBUF):
            pltpu.async_copy(in_slice(k0 + b), bufs[b], sems[b])

        def chunk_body(k, b):
            pltpu.make_async_copy(in_slice(k), bufs[b], sems[b]).wait()
            bb = bufs[b]

            def group(gr, carry):
                c0 = gr * LANES
                ms, idxs = [], []
                for j in range(8):
                    base_c = j * 8
                    m = bb[base_c, pl.ds(c0, LANES)]
                    idx = jnp.full((LANES,), base_c, jnp.int32)
                    for q in range(1, 8):
                        c = base_c + q
                        v = bb[c, pl.ds(c0, LANES)]
                        upd = v > m
                        m = jnp.where(upd, v, m)
                        idx = jnp.where(upd, c, idx)
                    ms.append(m)
                    idxs.append(idx)
                while len(ms) > 1:
                    nm, ni = [], []
                    for j in range(0, len(ms), 2):
                        upd = ms[j + 1] > ms[j]
                        nm.append(jnp.where(upd, ms[j + 1], ms[j]))
                        ni.append(jnp.where(upd, idxs[j + 1], idxs[j]))
                    ms, idxs = nm, ni
                obuf[pl.ds(c0, LANES)] = plsc.load_gather(alpha_v, [idxs[0]])
                return carry

            lax.fori_loop(0, SC_BB // LANES, group, 0)

            t = t0 + k // ncol
            b0 = (k % ncol) * SC_BB
            pltpu.sync_copy(
                obuf, out_hbm.at[pl.ds((t - t0) * B + b0, SC_BB)])

            nxt = k + NBUF

            @pl.when(nxt < k0 + per_w)
            def _():
                pltpu.async_copy(in_slice(nxt), bufs[b], sems[b])

        def ring_body(i, carry):
            for b in range(NBUF):
                chunk_body(k0 + i * NBUF + b, b)
            return carry

        lax.fori_loop(0, per_w // NBUF, ring_body, 0)

    return sc_decode(xT2, alphabet_codes)


def _tc_decode_call(xT, comb2, T, V, B):
    def tc_kernel(comb_ref, x_ref, o_ref):
        xb = x_ref[...]
        m = jnp.max(xb, axis=1, keepdims=True)
        keyf = jnp.where(xb == m, comb_ref[...][None, :, :], -jnp.inf)
        best = jnp.max(keyf, axis=1)
        o_ref[...] = (-best).astype(jnp.int32) & 255

    return pl.pallas_call(
        tc_kernel,
        grid=(TC_T // TC_TB, B // TC_BB),
        in_specs=[
            pl.BlockSpec((V, 1), lambda i, j: (0, 0)),
            pl.BlockSpec((TC_TB, V, TC_BB), lambda i, j: (i, 0, j)),
        ],
        out_specs=pl.BlockSpec((TC_TB, TC_BB), lambda i, j: (i, j)),
        out_shape=jax.ShapeDtypeStruct((TC_T, B), jnp.int32),
        compiler_params=pltpu.CompilerParams(
            dimension_semantics=("parallel", "parallel")),
    )(comb2, xT)


def kernel(x, alphabet_codes):
    B, T, V = x.shape
    sc_t = T - TC_T
    xT = jnp.transpose(x, (1, 2, 0))
    xT2 = xT.reshape(T * V, B)
    comb2 = -((jnp.arange(V, dtype=jnp.int32) << 8) | alphabet_codes
              ).astype(jnp.float32).reshape(V, 1)

    sc_out = _sc_decode_call(xT2, alphabet_codes, V, B, TC_T, sc_t)
    tc_out = _tc_decode_call(xT, comb2, T, V, B)
    oT = jnp.concatenate([tc_out, sc_out.reshape(sc_t, B)], axis=0)
    return jnp.transpose(oT)
